# baseline (speedup 1.0000x reference)
import functools

import jax
import jax.numpy as jnp
from jax import lax
from jax.experimental import pallas as pl
from jax.experimental.pallas import tpu as pltpu
from jax.experimental.pallas import tpu_sc as plsc

NUM_USERS = 25000
NUM_ITEMS = 25000
N = NUM_USERS + NUM_ITEMS
E = 800000
K = 64
BATCH = 4096

NC = 2
NS = 16
L = 16

HALF = N // NC
HALF_PAD = 25088
TRASH = HALF_PAD
SH_ROWS = 25104
ROWS_PER_TILE_OUT = HALF_PAD // NS

EDGES_PER_TILE = E // NS
BLOCK = 80
NBLK = EDGES_PER_TILE // BLOCK
NSLOT = 5

_MESH = plsc.VectorSubcoreMesh(core_axis_name="c", subcore_axis_name="s")


def _lap_body(ego_hbm, src_hbm, dst_hbm, val_hbm, out_hbm,
              src0, src1, src2, src3, src4,
              dst0, dst1, dst2, dst3, dst4,
              val0, val1, val2, val3, val4,
              idx0, idx1, idx2, idx3, idx4,
              rows0, rows1, rows2, rows3, rows4, lap_sh,
              lin0, lin1, lin2, lin3, lin4,
              g0, g1, g2, g3, g4, w0, w1, w2, w3, w4):
    c = lax.axis_index("c")
    s = lax.axis_index("s")
    base_row = c * HALF
    srcv = (src0, src1, src2, src3, src4)
    dstv = (dst0, dst1, dst2, dst3, dst4)
    valv = (val0, val1, val2, val3, val4)
    idx = (idx0, idx1, idx2, idx3, idx4)
    rows = (rows0, rows1, rows2, rows3, rows4)
    lin = (lin0, lin1, lin2, lin3, lin4)
    gsem = (g0, g1, g2, g3, g4)
    wsem = (w0, w1, w2, w3, w4)

    def _z(e, _):
        for sl in range(NSLOT):
            for j in range(K // L):
                rows[sl][e, pl.ds(j * L, L)] = jnp.zeros((L,), jnp.float32)
        return 0
    lax.fori_loop(0, BLOCK, _z, 0)
    zrow = s * ROWS_PER_TILE_OUT

    def _zf(i, _):
        pltpu.async_copy(rows0, lap_sh.at[pl.ds(zrow + i * BLOCK, BLOCK)], lin0)
        return 0
    lax.fori_loop(0, ROWS_PER_TILE_OUT // BLOCK, _zf, 0)
    pltpu.async_copy(rows1.at[pl.ds(0, 48)],
                     lap_sh.at[pl.ds(zrow + (ROWS_PER_TILE_OUT // BLOCK) * BLOCK,
                                     48)], lin0)

    def _zw(i, _):
        pltpu.make_async_copy(rows0, lap_sh.at[pl.ds(zrow, BLOCK)], lin0).wait()
        return 0
    lax.fori_loop(0, ROWS_PER_TILE_OUT // BLOCK, _zw, 0)
    pltpu.make_async_copy(rows1.at[pl.ds(0, 48)],
                          lap_sh.at[pl.ds(zrow, 48)], lin0).wait()
    plsc.subcore_barrier()

    tile_base = s * EDGES_PER_TILE

    def _lin_start(b, slot):
        base = tile_base + b * BLOCK
        pltpu.async_copy(src_hbm.at[pl.ds(base, BLOCK)], srcv[slot], lin[slot])
        pltpu.async_copy(dst_hbm.at[pl.ds(base, BLOCK)], dstv[slot], lin[slot])
        pltpu.async_copy(val_hbm.at[pl.ds(base, BLOCK)], valv[slot], lin[slot])

    def _lin_wait(slot):
        pltpu.make_async_copy(src_hbm.at[pl.ds(0, BLOCK)], srcv[slot],
                              lin[slot]).wait()
        pltpu.make_async_copy(dst_hbm.at[pl.ds(0, BLOCK)], dstv[slot],
                              lin[slot]).wait()
        pltpu.make_async_copy(val_hbm.at[pl.ds(0, BLOCK)], valv[slot],
                              lin[slot]).wait()

    def _gather_start(slot):
        pltpu.async_copy(ego_hbm.at[srcv[slot]], rows[slot], gsem[slot])

    def _gather_wait(slot):
        pltpu.make_async_copy(ego_hbm.at[srcv[slot]], rows[slot],
                              gsem[slot]).wait()

    def _scatter_start(slot):
        pltpu.async_copy(rows[slot], lap_sh.at[idx[slot]], wsem[slot],
                         add=True)

    def _scatter_wait(slot):
        pltpu.make_async_copy(rows[slot], lap_sh.at[idx[slot]],
                              wsem[slot]).wait()

    def _xcompute(slot):
        for g in range(BLOCK // L):
            d = dstv[slot][pl.ds(g * L, L)]
            loc = d - base_row
            ok = (d >= base_row) & (loc < HALF)
            idx[slot][pl.ds(g * L, L)] = jnp.where(ok, loc, TRASH)

    def _scale(slot):
        def _sg(g, _):
            vals16 = valv[slot][pl.ds(g * L, L)]
            for l in range(L):
                e = g * L + l
                v = jnp.full((L,), vals16[l], jnp.float32)
                for j in range(K // L):
                    rows[slot][e, pl.ds(j * L, L)] = (
                        rows[slot][e, pl.ds(j * L, L)] * v)
            return 0
        lax.fori_loop(0, BLOCK // L, _sg, 0)

    def _step(b, A, drain=True, prefetch=True, fire_lin=True):
        D = (A + 4) % NSLOT
        _gather_wait(A)
        if drain:
            _scatter_wait(D)
        if prefetch:
            _lin_wait(D)
            _gather_start(D)
        _xcompute(A)
        _scale(A)
        _scatter_start(A)
        if fire_lin:
            _lin_start(b + NSLOT, A)

    for b in range(NSLOT):
        _lin_start(b, b)
    for b in range(NSLOT - 1):
        _lin_wait(b)
        _gather_start(b)

    _step(0, 0, drain=False)
    def _quint(i, _):
        b = 5 * i + 1
        for q in range(NSLOT):
            _step(b + q, (1 + q) % NSLOT)
        return 0
    lax.fori_loop(0, 123, _quint, 0)
    for b in range(616, 620):
        _step(b, b % NSLOT)
    _step(620, 0, fire_lin=False)
    for b in range(621, NBLK):
        _step(b, b % NSLOT, prefetch=False, fire_lin=False)
    _scatter_wait((NBLK - 1) % NSLOT)

    plsc.subcore_barrier()

    orow = s * ROWS_PER_TILE_OUT
    pltpu.sync_copy(lap_sh.at[pl.ds(orow, ROWS_PER_TILE_OUT)],
                    out_hbm.at[c, pl.ds(orow, ROWS_PER_TILE_OUT)])


_lap_call = pl.kernel(
    _lap_body,
    out_type=jax.ShapeDtypeStruct((NC, HALF_PAD, K), jnp.float32),
    mesh=_MESH,
    scratch_types=(
        [pltpu.VMEM((BLOCK,), jnp.int32) for _ in range(5)] +
        [pltpu.VMEM((BLOCK,), jnp.int32) for _ in range(5)] +
        [pltpu.VMEM((BLOCK,), jnp.float32) for _ in range(5)] +
        [pltpu.VMEM((BLOCK,), jnp.int32) for _ in range(5)] +
        [pltpu.VMEM((BLOCK, K), jnp.float32) for _ in range(5)] +
        [pltpu.VMEM_SHARED((SH_ROWS, K), jnp.float32)] +
        [pltpu.SemaphoreType.DMA for _ in range(15)]
    ),
    compiler_params=pltpu.CompilerParams(use_tc_tiling_on_sc=False),
)


DBLK = 200
DGRID = N // DBLK


def _dense_body(lap_ref, ego_ref, W1_ref, b1_ref, W2_ref, b2_ref,
                ego_out_ref, norm_out_ref):
    lap = lap_ref[0]
    ego = ego_ref[...]
    first = jnp.dot(lap + ego, W1_ref[...],
                    preferred_element_type=jnp.float32) + b1_ref[...]
    second = jnp.dot(ego * lap, W2_ref[...],
                     preferred_element_type=jnp.float32) + b2_ref[...]
    x = first + second
    x = jnp.where(x >= 0, x, 0.2 * x)
    ego_out_ref[...] = x
    ss = jnp.sum(x * x, axis=1, keepdims=True)
    norm_out_ref[...] = x * lax.rsqrt(jnp.maximum(ss, 1e-12))


def _dense_call(lap2, ego, W1, b1, W2, b2):
    per_half = HALF // DBLK
    return pl.pallas_call(
        _dense_body,
        grid=(DGRID,),
        in_specs=[
            pl.BlockSpec((1, DBLK, K), lambda i: (i // per_half, i % per_half, 0)),
            pl.BlockSpec((DBLK, K), lambda i: (i, 0)),
            pl.BlockSpec((K, K), lambda i: (0, 0)),
            pl.BlockSpec((1, K), lambda i: (0, 0)),
            pl.BlockSpec((K, K), lambda i: (0, 0)),
            pl.BlockSpec((1, K), lambda i: (0, 0)),
        ],
        out_specs=[
            pl.BlockSpec((DBLK, K), lambda i: (i, 0)),
            pl.BlockSpec((DBLK, K), lambda i: (i, 0)),
        ],
        out_shape=[
            jax.ShapeDtypeStruct((N, K), jnp.float32),
            jax.ShapeDtypeStruct((N, K), jnp.float32),
        ],
    )(lap2, ego, W1, b1, W2, b2)


BPT = BATCH // (NC * NS)


def _lookup_body(e0_hbm, n1_hbm, n2_hbm, user_hbm, item_hbm,
                 gu_hbm, gi_hbm,
                 u_idx, i_idx, bufs_u, bufs_i, sem):
    c = lax.axis_index("c")
    s = lax.axis_index("s")
    wid = s * NC + c
    base = wid * BPT

    pltpu.sync_copy(user_hbm.at[pl.ds(base, BPT)], u_idx)
    pltpu.sync_copy(item_hbm.at[pl.ds(base, BPT)], i_idx)
    for g in range(BPT // L):
        i_idx[pl.ds(g * L, L)] = i_idx[pl.ds(g * L, L)] + NUM_USERS

    for t, tbl in enumerate((e0_hbm, n1_hbm, n2_hbm)):
        pltpu.async_copy(tbl.at[u_idx], bufs_u.at[t], sem).wait()
        pltpu.async_copy(tbl.at[i_idx], bufs_i.at[t], sem).wait()

    for t in range(3):
        pltpu.sync_copy(bufs_u.at[t], gu_hbm.at[t, pl.ds(base, BPT)])
        pltpu.sync_copy(bufs_i.at[t], gi_hbm.at[t, pl.ds(base, BPT)])


_lookup_call = pl.kernel(
    _lookup_body,
    out_type=(
        jax.ShapeDtypeStruct((3, BATCH, K), jnp.float32),
        jax.ShapeDtypeStruct((3, BATCH, K), jnp.float32),
    ),
    mesh=_MESH,
    scratch_types=[
        pltpu.VMEM((BPT,), jnp.int32),
        pltpu.VMEM((BPT,), jnp.int32),
        pltpu.VMEM((3, BPT, K), jnp.float32),
        pltpu.VMEM((3, BPT, K), jnp.float32),
        pltpu.SemaphoreType.DMA,
    ],
    compiler_params=pltpu.CompilerParams(use_tc_tiling_on_sc=False),
)


def _xui_body(gu3_ref, gi3_ref, gu_ref, gi_ref, xui_ref):
    gu = jnp.concatenate([gu3_ref[0], gu3_ref[1], gu3_ref[2]], axis=1)
    gi = jnp.concatenate([gi3_ref[0], gi3_ref[1], gi3_ref[2]], axis=1)
    gu_ref[...] = gu
    gi_ref[...] = gi
    xui_ref[...] = jnp.sum(gu * gi, axis=1)


_xui_call = pl.pallas_call(
    _xui_body,
    out_shape=(
        jax.ShapeDtypeStruct((BATCH, 3 * K), jnp.float32),
        jax.ShapeDtypeStruct((BATCH, 3 * K), jnp.float32),
        jax.ShapeDtypeStruct((BATCH,), jnp.float32),
    ),
)


def kernel(gu0, gi0, edge_vals, W1_0, b1_0, W2_0, b2_0, W1_1, b1_1, W2_1, b2_1,
           user, item, edge_index):
    ego0 = jnp.concatenate([gu0, gi0], axis=0)
    src_ = edge_index[0]
    dst_ = edge_index[1]

    lap1 = _lap_call(ego0, src_, dst_, edge_vals)
    ego1, norm1 = _dense_call(lap1, ego0, W1_0, b1_0, W2_0, b2_0)
    lap2 = _lap_call(ego1, src_, dst_, edge_vals)
    _, norm2 = _dense_call(lap2, ego1, W1_1, b1_1, W2_1, b2_1)

    gu3, gi3 = _lookup_call(ego0, norm1, norm2, user, item)
    gamma_u, gamma_i, xui = _xui_call(gu3, gi3)
    return (xui, gamma_u, gamma_i)


# speedup vs baseline: 1.0562x; 1.0562x over previous
import functools

import jax
import jax.numpy as jnp
from jax import lax
from jax.experimental import pallas as pl
from jax.experimental.pallas import tpu as pltpu
from jax.experimental.pallas import tpu_sc as plsc

NUM_USERS = 25000
NUM_ITEMS = 25000
N = NUM_USERS + NUM_ITEMS
E = 800000
K = 64
BATCH = 4096

NC = 2
NS = 16
L = 16

HALF = N // NC
HALF_PAD = 25088
TRASH = HALF_PAD
SH_ROWS = 25104
ROWS_PER_TILE_OUT = HALF_PAD // NS

EDGES_PER_TILE = E // NS
BLOCK = 128
EPT_PAD = 50048
NBLK = EPT_PAD // BLOCK
NSLOT = 3

_MESH = plsc.VectorSubcoreMesh(core_axis_name="c", subcore_axis_name="s")


def _lap_body(ego_hbm, src_hbm, dst_hbm, val_hbm, out_hbm,
              src0, src1, src2, dst0, dst1, dst2, val0, val1, val2,
              idx0, idx1, idx2, rows0, rows1, rows2, lap_sh,
              lin0, lin1, lin2, g0, g1, g2, w0, w1, w2):
    c = lax.axis_index("c")
    s = lax.axis_index("s")
    base_row = c * HALF
    srcv = (src0, src1, src2)
    dstv = (dst0, dst1, dst2)
    valv = (val0, val1, val2)
    idx = (idx0, idx1, idx2)
    rows = (rows0, rows1, rows2)
    lin = (lin0, lin1, lin2)
    gsem = (g0, g1, g2)
    wsem = (w0, w1, w2)

    def _z(e, _):
        for sl in range(NSLOT):
            for j in range(K // L):
                rows[sl][e, pl.ds(j * L, L)] = jnp.zeros((L,), jnp.float32)
        return 0
    lax.fori_loop(0, BLOCK, _z, 0)
    zrow = s * ROWS_PER_TILE_OUT

    def _zf(i, _):
        pltpu.async_copy(rows0, lap_sh.at[pl.ds(zrow + i * BLOCK, BLOCK)], lin0)
        return 0
    lax.fori_loop(0, ROWS_PER_TILE_OUT // BLOCK, _zf, 0)
    pltpu.async_copy(rows1.at[pl.ds(0, 32)],
                     lap_sh.at[pl.ds(zrow + (ROWS_PER_TILE_OUT // BLOCK) * BLOCK,
                                     32)], lin0)

    def _zw(i, _):
        pltpu.make_async_copy(rows0, lap_sh.at[pl.ds(zrow, BLOCK)], lin0).wait()
        return 0
    lax.fori_loop(0, ROWS_PER_TILE_OUT // BLOCK, _zw, 0)
    pltpu.make_async_copy(rows1.at[pl.ds(0, 32)],
                          lap_sh.at[pl.ds(zrow, 32)], lin0).wait()
    plsc.subcore_barrier()

    def _lin_start(b, slot):
        base = b * BLOCK
        pltpu.async_copy(src_hbm.at[s, pl.ds(base, BLOCK)], srcv[slot], lin[slot])
        pltpu.async_copy(dst_hbm.at[s, pl.ds(base, BLOCK)], dstv[slot], lin[slot])
        pltpu.async_copy(val_hbm.at[s, pl.ds(base, BLOCK)], valv[slot], lin[slot])

    def _lin_wait(slot):
        pltpu.make_async_copy(src_hbm.at[0, pl.ds(0, BLOCK)], srcv[slot],
                              lin[slot]).wait()
        pltpu.make_async_copy(dst_hbm.at[0, pl.ds(0, BLOCK)], dstv[slot],
                              lin[slot]).wait()
        pltpu.make_async_copy(val_hbm.at[0, pl.ds(0, BLOCK)], valv[slot],
                              lin[slot]).wait()

    def _gather_start(slot):
        pltpu.async_copy(ego_hbm.at[srcv[slot]], rows[slot], gsem[slot])

    def _gather_wait(slot):
        pltpu.make_async_copy(ego_hbm.at[srcv[slot]], rows[slot],
                              gsem[slot]).wait()

    def _scatter_start(slot):
        pltpu.async_copy(rows[slot], lap_sh.at[idx[slot]], wsem[slot],
                         add=True)

    def _scatter_wait(slot):
        pltpu.make_async_copy(rows[slot], lap_sh.at[idx[slot]],
                              wsem[slot]).wait()

    def _xcompute(slot):
        for g in range(BLOCK // L):
            d = dstv[slot][pl.ds(g * L, L)]
            loc = d - base_row
            ok = (d >= base_row) & (loc < HALF)
            idx[slot][pl.ds(g * L, L)] = jnp.where(ok, loc, TRASH)

    def _scale(slot):
        def _sg(g, _):
            vals16 = valv[slot][pl.ds(g * L, L)]
            for l in range(L):
                e = g * L + l
                v = jnp.full((L,), vals16[l], jnp.float32)
                for j in range(K // L):
                    rows[slot][e, pl.ds(j * L, L)] = (
                        rows[slot][e, pl.ds(j * L, L)] * v)
            return 0
        lax.fori_loop(0, BLOCK // L, _sg, 0)

    def _step(b, A, drain=True, prefetch=True, fire_lin=True):
        D = (A + 2) % NSLOT
        _gather_wait(A)
        if drain:
            _scatter_wait(D)
        if prefetch:
            _lin_wait(D)
            _gather_start(D)
        _xcompute(A)
        _scale(A)
        _scatter_start(A)
        if fire_lin:
            _lin_start(b + NSLOT, A)

    for b in range(NSLOT):
        _lin_start(b, b)
    for b in range(NSLOT - 1):
        _lin_wait(b)
        _gather_start(b)

    _step(0, 0, drain=False)
    def _trip(i, _):
        b = 3 * i + 1
        for q in range(NSLOT):
            _step(b + q, (1 + q) % NSLOT)
        return 0
    lax.fori_loop(0, 129, _trip, 0)
    _step(NBLK - 3, (NBLK - 3) % NSLOT, fire_lin=False)
    _step(NBLK - 2, (NBLK - 2) % NSLOT, fire_lin=False, prefetch=False)
    _step(NBLK - 1, (NBLK - 1) % NSLOT, fire_lin=False, prefetch=False)
    _scatter_wait((NBLK - 1) % NSLOT)

    plsc.subcore_barrier()

    orow = s * ROWS_PER_TILE_OUT
    pltpu.sync_copy(lap_sh.at[pl.ds(orow, ROWS_PER_TILE_OUT)],
                    out_hbm.at[c, pl.ds(orow, ROWS_PER_TILE_OUT)])


_lap_call = pl.kernel(
    _lap_body,
    out_type=jax.ShapeDtypeStruct((NC, HALF_PAD, K), jnp.float32),
    mesh=_MESH,
    scratch_types=(
        [pltpu.VMEM((BLOCK,), jnp.int32) for _ in range(3)] +
        [pltpu.VMEM((BLOCK,), jnp.int32) for _ in range(3)] +
        [pltpu.VMEM((BLOCK,), jnp.float32) for _ in range(3)] +
        [pltpu.VMEM((BLOCK,), jnp.int32) for _ in range(3)] +
        [pltpu.VMEM((BLOCK, K), jnp.float32) for _ in range(3)] +
        [pltpu.VMEM_SHARED((SH_ROWS, K), jnp.float32)] +
        [pltpu.SemaphoreType.DMA for _ in range(9)]
    ),
    compiler_params=pltpu.CompilerParams(use_tc_tiling_on_sc=False),
)


DBLK = 200
DGRID = N // DBLK


def _dense_body(lap_ref, ego_ref, W1_ref, b1_ref, W2_ref, b2_ref,
                ego_out_ref, norm_out_ref):
    lap = lap_ref[0]
    ego = ego_ref[...]
    first = jnp.dot(lap + ego, W1_ref[...],
                    preferred_element_type=jnp.float32) + b1_ref[...]
    second = jnp.dot(ego * lap, W2_ref[...],
                     preferred_element_type=jnp.float32) + b2_ref[...]
    x = first + second
    x = jnp.where(x >= 0, x, 0.2 * x)
    ego_out_ref[...] = x
    ss = jnp.sum(x * x, axis=1, keepdims=True)
    norm_out_ref[...] = x * lax.rsqrt(jnp.maximum(ss, 1e-12))


def _dense_call(lap2, ego, W1, b1, W2, b2):
    per_half = HALF // DBLK
    return pl.pallas_call(
        _dense_body,
        grid=(DGRID,),
        in_specs=[
            pl.BlockSpec((1, DBLK, K), lambda i: (i // per_half, i % per_half, 0)),
            pl.BlockSpec((DBLK, K), lambda i: (i, 0)),
            pl.BlockSpec((K, K), lambda i: (0, 0)),
            pl.BlockSpec((1, K), lambda i: (0, 0)),
            pl.BlockSpec((K, K), lambda i: (0, 0)),
            pl.BlockSpec((1, K), lambda i: (0, 0)),
        ],
        out_specs=[
            pl.BlockSpec((DBLK, K), lambda i: (i, 0)),
            pl.BlockSpec((DBLK, K), lambda i: (i, 0)),
        ],
        out_shape=[
            jax.ShapeDtypeStruct((N, K), jnp.float32),
            jax.ShapeDtypeStruct((N, K), jnp.float32),
        ],
    )(lap2, ego, W1, b1, W2, b2)


BPT = BATCH // (NC * NS)


def _lookup_body(e0_hbm, n1_hbm, n2_hbm, user_hbm, item_hbm,
                 gu_hbm, gi_hbm,
                 u_idx, i_idx, bufs_u, bufs_i, sem):
    c = lax.axis_index("c")
    s = lax.axis_index("s")
    wid = s * NC + c
    base = wid * BPT

    pltpu.sync_copy(user_hbm.at[pl.ds(base, BPT)], u_idx)
    pltpu.sync_copy(item_hbm.at[pl.ds(base, BPT)], i_idx)
    for g in range(BPT // L):
        i_idx[pl.ds(g * L, L)] = i_idx[pl.ds(g * L, L)] + NUM_USERS

    for t, tbl in enumerate((e0_hbm, n1_hbm, n2_hbm)):
        pltpu.async_copy(tbl.at[u_idx], bufs_u.at[t], sem).wait()
        pltpu.async_copy(tbl.at[i_idx], bufs_i.at[t], sem).wait()

    for t in range(3):
        pltpu.sync_copy(bufs_u.at[t], gu_hbm.at[t, pl.ds(base, BPT)])
        pltpu.sync_copy(bufs_i.at[t], gi_hbm.at[t, pl.ds(base, BPT)])


_lookup_call = pl.kernel(
    _lookup_body,
    out_type=(
        jax.ShapeDtypeStruct((3, BATCH, K), jnp.float32),
        jax.ShapeDtypeStruct((3, BATCH, K), jnp.float32),
    ),
    mesh=_MESH,
    scratch_types=[
        pltpu.VMEM((BPT,), jnp.int32),
        pltpu.VMEM((BPT,), jnp.int32),
        pltpu.VMEM((3, BPT, K), jnp.float32),
        pltpu.VMEM((3, BPT, K), jnp.float32),
        pltpu.SemaphoreType.DMA,
    ],
    compiler_params=pltpu.CompilerParams(use_tc_tiling_on_sc=False),
)


def _xui_body(gu3_ref, gi3_ref, gu_ref, gi_ref, xui_ref):
    gu = jnp.concatenate([gu3_ref[0], gu3_ref[1], gu3_ref[2]], axis=1)
    gi = jnp.concatenate([gi3_ref[0], gi3_ref[1], gi3_ref[2]], axis=1)
    gu_ref[...] = gu
    gi_ref[...] = gi
    xui_ref[...] = jnp.sum(gu * gi, axis=1)


_xui_call = pl.pallas_call(
    _xui_body,
    out_shape=(
        jax.ShapeDtypeStruct((BATCH, 3 * K), jnp.float32),
        jax.ShapeDtypeStruct((BATCH, 3 * K), jnp.float32),
        jax.ShapeDtypeStruct((BATCH,), jnp.float32),
    ),
)


def kernel(gu0, gi0, edge_vals, W1_0, b1_0, W2_0, b2_0, W1_1, b1_1, W2_1, b2_1,
           user, item, edge_index):
    ego0 = jnp.concatenate([gu0, gi0], axis=0)
    pad = EPT_PAD - EDGES_PER_TILE
    src_t = jnp.pad(edge_index[0].reshape(NS, EDGES_PER_TILE),
                    ((0, 0), (0, pad)), constant_values=0)
    dst_t = jnp.pad(edge_index[1].reshape(NS, EDGES_PER_TILE),
                    ((0, 0), (0, pad)), constant_values=N)
    val_t = jnp.pad(edge_vals.reshape(NS, EDGES_PER_TILE),
                    ((0, 0), (0, pad)), constant_values=0.0)

    lap1 = _lap_call(ego0, src_t, dst_t, val_t)
    ego1, norm1 = _dense_call(lap1, ego0, W1_0, b1_0, W2_0, b2_0)
    lap2 = _lap_call(ego1, src_t, dst_t, val_t)
    _, norm2 = _dense_call(lap2, ego1, W1_1, b1_1, W2_1, b2_1)

    gu3, gi3 = _lookup_call(ego0, norm1, norm2, user, item)
    gamma_u, gamma_i, xui = _xui_call(gu3, gi3)
    return (xui, gamma_u, gamma_i)
